# baseline (speedup 1.0000x reference)
import functools

import jax
import jax.numpy as jnp
from jax import lax
from jax.experimental import pallas as pl
from jax.experimental.pallas import tpu as pltpu
from jax.experimental.pallas import tpu_sc as plsc

N = 8192
F_IN = 128
F_PRIME = 64
NEG_SLOPE = 0.01

ROW_BLOCK = 512
SC_COLS = 2048
TC_COLS = N - SC_COLS
NW = 32
ROWS_PER_W = N // NW
RB = 16
LAN = 16


def _proj_kernel(h_ref, wst_ref, bs_ref, wtt_ref, bt_ref, a1_ref, a2_ref, ab_ref,
                 z_ref, P_ref, psm_ref, Q_ref, qsm_ref):
    h = h_ref[...]
    z_src = jnp.dot(h, wst_ref[...], preferred_element_type=jnp.float32) + bs_ref[...]
    z_tar = jnp.dot(h, wtt_ref[...], preferred_element_type=jnp.float32) + bt_ref[...]
    s = jnp.dot(z_src, a1_ref[...], preferred_element_type=jnp.float32) + ab_ref[...]
    t = jnp.dot(z_tar, a2_ref[...], preferred_element_type=jnp.float32)
    z_ref[...] = z_src
    P_ref[...] = jnp.exp(s)
    psm_ref[...] = jnp.exp(NEG_SLOPE * s)
    Q_ref[...] = jnp.exp(t)
    qsm_ref[...] = jnp.exp(NEG_SLOPE * t)


def _sc_den_kernel(a_hbm, P_hbm, p_hbm, Qt_hbm, qt_hbm, out_hbm,
                   qv, qqv, Pv, pv, abuf, dwide):
    c = lax.axis_index("c")
    s = lax.axis_index("s")
    wid = s * 2 + c
    row0 = wid * ROWS_PER_W
    pltpu.sync_copy(Qt_hbm, qv)
    pltpu.sync_copy(qt_hbm, qqv)
    pltpu.sync_copy(P_hbm.at[pl.ds(row0, ROWS_PER_W)], Pv)
    pltpu.sync_copy(p_hbm.at[pl.ds(row0, ROWS_PER_W)], pv)

    def blk_body(blk, carry):
        r0 = blk * RB
        pltpu.sync_copy(
            a_hbm.at[pl.ds(row0 + r0, RB), pl.ds(TC_COLS, SC_COLS)], abuf)
        Pgrp = Pv[pl.ds(r0, RB)]
        pgrp = pv[pl.ds(r0, RB)]
        for rr in range(RB):
            rl = r0 + rr
            Pb = jnp.full((LAN,), Pgrp[rr], jnp.float32)
            pb = jnp.full((LAN,), pgrp[rr], jnp.float32)

            def k_body(k, acc):
                off = k * LAN
                a16 = abuf[rr, pl.ds(off, LAN)].astype(jnp.float32)
                e16 = jnp.maximum(Pb * qv[pl.ds(off, LAN)],
                                  pb * qqv[pl.ds(off, LAN)])
                return acc + a16 * e16

            acc = lax.fori_loop(0, SC_COLS // LAN, k_body,
                                jnp.zeros((LAN,), jnp.float32))
            dwide[pl.ds(rl * LAN, LAN)] = acc
        return carry

    lax.fori_loop(0, ROWS_PER_W // RB, blk_body, jnp.int32(0))
    pltpu.sync_copy(dwide, out_hbm.at[pl.ds(row0 * LAN, ROWS_PER_W * LAN)])


def _agg_kernel(a_ref, P_ref, psm_ref, Q_ref, qsm_ref, z_ref, num_ref, den_ref):
    e = jnp.maximum(P_ref[...] * Q_ref[...], psm_ref[...] * qsm_ref[...])
    den_ref[...] = jnp.sum(e[:, :TC_COLS] * a_ref[...].astype(jnp.float32),
                           axis=1, keepdims=True)
    num_ref[...] = jnp.dot(e, z_ref[...], preferred_element_type=jnp.float32)


def _combine_kernel(num_ref, dtc_ref, dsc_ref, out_ref):
    den = dtc_ref[...] + jnp.sum(dsc_ref[...], axis=1, keepdims=True)
    out_ref[...] = jax.nn.sigmoid(num_ref[...] / den)


@jax.jit
def kernel(H, A, W_src_w, W_src_b, W_tar_w, W_tar_b, a_w, a_b):
    wst = W_src_w.T
    wtt = W_tar_w.T
    bs = W_src_b.reshape(1, F_PRIME)
    bt = W_tar_b.reshape(1, F_PRIME)
    a1 = a_w[:, :F_PRIME].T
    a2 = a_w[:, F_PRIME:].T
    ab = a_b.reshape(1, 1)

    z_src, P, p_sm, Q, q_sm = pl.pallas_call(
        _proj_kernel,
        out_shape=(
            jax.ShapeDtypeStruct((N, F_PRIME), jnp.float32),
            jax.ShapeDtypeStruct((N, 1), jnp.float32),
            jax.ShapeDtypeStruct((N, 1), jnp.float32),
            jax.ShapeDtypeStruct((N, 1), jnp.float32),
            jax.ShapeDtypeStruct((N, 1), jnp.float32),
        ),
    )(H, wst, bs, wtt, bt, a1, a2, ab)

    Q_row = Q.T
    q_row = q_sm.T
    P_1d = P.reshape(N)
    p_1d = p_sm.reshape(N)
    Qt = Q_row.reshape(N)[TC_COLS:]
    qt = q_row.reshape(N)[TC_COLS:]

    sc_mesh = plsc.VectorSubcoreMesh(core_axis_name="c", subcore_axis_name="s",
                                     num_cores=2, num_subcores=16)
    den_sc_flat = pl.kernel(
        _sc_den_kernel,
        out_type=jax.ShapeDtypeStruct((N * LAN,), jnp.float32),
        mesh=sc_mesh,
        scratch_types=[
            pltpu.VMEM((SC_COLS,), jnp.float32),
            pltpu.VMEM((SC_COLS,), jnp.float32),
            pltpu.VMEM((ROWS_PER_W,), jnp.float32),
            pltpu.VMEM((ROWS_PER_W,), jnp.float32),
            pltpu.VMEM((RB, SC_COLS), jnp.int32),
            pltpu.VMEM((ROWS_PER_W * LAN,), jnp.float32),
        ],
    )(A, P_1d, p_1d, Qt, qt)

    grid = (N // ROW_BLOCK,)
    num, den_tc = pl.pallas_call(
        _agg_kernel,
        grid=grid,
        in_specs=[
            pl.BlockSpec((ROW_BLOCK, TC_COLS), lambda i: (i, 0)),
            pl.BlockSpec((ROW_BLOCK, 1), lambda i: (i, 0)),
            pl.BlockSpec((ROW_BLOCK, 1), lambda i: (i, 0)),
            pl.BlockSpec((1, N), lambda i: (0, 0)),
            pl.BlockSpec((1, N), lambda i: (0, 0)),
            pl.BlockSpec((N, F_PRIME), lambda i: (0, 0)),
        ],
        out_specs=(
            pl.BlockSpec((ROW_BLOCK, F_PRIME), lambda i: (i, 0)),
            pl.BlockSpec((ROW_BLOCK, 1), lambda i: (i, 0)),
        ),
        out_shape=(
            jax.ShapeDtypeStruct((N, F_PRIME), jnp.float32),
            jax.ShapeDtypeStruct((N, 1), jnp.float32),
        ),
    )(A, P, p_sm, Q_row, q_row, z_src)

    den_sc = den_sc_flat.reshape(N, LAN)
    out = pl.pallas_call(
        _combine_kernel,
        out_shape=jax.ShapeDtypeStruct((N, F_PRIME), jnp.float32),
    )(num, den_tc, den_sc)
    return out


# speedup vs baseline: 1.3716x; 1.3716x over previous
import functools

import jax
import jax.numpy as jnp
from jax import lax
from jax.experimental import pallas as pl
from jax.experimental.pallas import tpu as pltpu
from jax.experimental.pallas import tpu_sc as plsc

N = 8192
F_IN = 128
F_PRIME = 64
NEG_SLOPE = 0.01

ROW_BLOCK = 512
SC_COLS = 2048
TC_COLS = N - SC_COLS
NW = 32
ROWS_PER_W = N // NW
RB = 16
LAN = 16


def _proj_kernel(h_ref, wst_ref, bs_ref, wtt_ref, bt_ref, a1_ref, a2_ref, ab_ref,
                 z_ref, P_ref, psm_ref, Q_ref, qsm_ref):
    h = h_ref[...]
    z_src = jnp.dot(h, wst_ref[...], preferred_element_type=jnp.float32) + bs_ref[...]
    z_tar = jnp.dot(h, wtt_ref[...], preferred_element_type=jnp.float32) + bt_ref[...]
    s = jnp.dot(z_src, a1_ref[...], preferred_element_type=jnp.float32) + ab_ref[...]
    t = jnp.dot(z_tar, a2_ref[...], preferred_element_type=jnp.float32)
    z_ref[...] = z_src
    P_ref[...] = jnp.exp(s)
    psm_ref[...] = jnp.exp(NEG_SLOPE * s)
    Q_ref[...] = jnp.exp(t)
    qsm_ref[...] = jnp.exp(NEG_SLOPE * t)


def _sc_den_kernel(a_hbm, P_hbm, p_hbm, Qt_hbm, qt_hbm, out_hbm,
                   qv, qqv, Pv, pv, abuf, dwide):
    c = lax.axis_index("c")
    s = lax.axis_index("s")
    wid = s * 2 + c
    row0 = wid * ROWS_PER_W
    pltpu.sync_copy(Qt_hbm, qv)
    pltpu.sync_copy(qt_hbm, qqv)
    pltpu.sync_copy(P_hbm.at[pl.ds(row0, ROWS_PER_W)], Pv)
    pltpu.sync_copy(p_hbm.at[pl.ds(row0, ROWS_PER_W)], pv)

    def blk_body(blk, carry):
        r0 = blk * RB
        pltpu.sync_copy(
            a_hbm.at[pl.ds(row0 + r0, RB), pl.ds(TC_COLS, SC_COLS)], abuf)
        Pgrp = Pv[pl.ds(r0, RB)]
        pgrp = pv[pl.ds(r0, RB)]
        Pbs = [jnp.full((LAN,), Pgrp[rr], jnp.float32) for rr in range(RB)]
        pbs = [jnp.full((LAN,), pgrp[rr], jnp.float32) for rr in range(RB)]
        zero = jnp.zeros((LAN,), jnp.float32)

        @plsc.parallel_loop(0, SC_COLS // LAN, carry=(zero,) * RB)
        def k_loop(k, accs):
            off = k * LAN
            qk = qv[pl.ds(off, LAN)]
            qqk = qqv[pl.ds(off, LAN)]
            new = []
            for rr in range(RB):
                a16 = abuf[rr, pl.ds(off, LAN)].astype(jnp.float32)
                e16 = jnp.maximum(Pbs[rr] * qk, pbs[rr] * qqk)
                new.append(accs[rr] + a16 * e16)
            return tuple(new)

        for rr in range(RB):
            dwide[pl.ds((r0 + rr) * LAN, LAN)] = k_loop[rr]
        return carry

    lax.fori_loop(0, ROWS_PER_W // RB, blk_body, jnp.int32(0))
    pltpu.sync_copy(dwide, out_hbm.at[pl.ds(row0 * LAN, ROWS_PER_W * LAN)])


def _agg_kernel(a_ref, P_ref, psm_ref, Q_ref, qsm_ref, z_ref, num_ref, den_ref):
    e = jnp.maximum(P_ref[...] * Q_ref[...], psm_ref[...] * qsm_ref[...])
    den_ref[...] = jnp.sum(e[:, :TC_COLS] * a_ref[...].astype(jnp.float32),
                           axis=1, keepdims=True)
    num_ref[...] = jnp.dot(e, z_ref[...], preferred_element_type=jnp.float32)


def _combine_kernel(num_ref, dtc_ref, dsc_ref, out_ref):
    den = dtc_ref[...] + jnp.sum(dsc_ref[...], axis=1, keepdims=True)
    out_ref[...] = jax.nn.sigmoid(num_ref[...] / den)


@jax.jit
def kernel(H, A, W_src_w, W_src_b, W_tar_w, W_tar_b, a_w, a_b):
    wst = W_src_w.T
    wtt = W_tar_w.T
    bs = W_src_b.reshape(1, F_PRIME)
    bt = W_tar_b.reshape(1, F_PRIME)
    a1 = a_w[:, :F_PRIME].T
    a2 = a_w[:, F_PRIME:].T
    ab = a_b.reshape(1, 1)

    z_src, P, p_sm, Q, q_sm = pl.pallas_call(
        _proj_kernel,
        out_shape=(
            jax.ShapeDtypeStruct((N, F_PRIME), jnp.float32),
            jax.ShapeDtypeStruct((N, 1), jnp.float32),
            jax.ShapeDtypeStruct((N, 1), jnp.float32),
            jax.ShapeDtypeStruct((N, 1), jnp.float32),
            jax.ShapeDtypeStruct((N, 1), jnp.float32),
        ),
    )(H, wst, bs, wtt, bt, a1, a2, ab)

    Q_row = Q.T
    q_row = q_sm.T
    P_1d = P.reshape(N)
    p_1d = p_sm.reshape(N)
    Qt = Q_row.reshape(N)[TC_COLS:]
    qt = q_row.reshape(N)[TC_COLS:]

    sc_mesh = plsc.VectorSubcoreMesh(core_axis_name="c", subcore_axis_name="s",
                                     num_cores=2, num_subcores=16)
    den_sc_flat = pl.kernel(
        _sc_den_kernel,
        out_type=jax.ShapeDtypeStruct((N * LAN,), jnp.float32),
        mesh=sc_mesh,
        scratch_types=[
            pltpu.VMEM((SC_COLS,), jnp.float32),
            pltpu.VMEM((SC_COLS,), jnp.float32),
            pltpu.VMEM((ROWS_PER_W,), jnp.float32),
            pltpu.VMEM((ROWS_PER_W,), jnp.float32),
            pltpu.VMEM((RB, SC_COLS), jnp.int32),
            pltpu.VMEM((ROWS_PER_W * LAN,), jnp.float32),
        ],
    )(A, P_1d, p_1d, Qt, qt)

    grid = (N // ROW_BLOCK,)
    num, den_tc = pl.pallas_call(
        _agg_kernel,
        grid=grid,
        in_specs=[
            pl.BlockSpec((ROW_BLOCK, TC_COLS), lambda i: (i, 0)),
            pl.BlockSpec((ROW_BLOCK, 1), lambda i: (i, 0)),
            pl.BlockSpec((ROW_BLOCK, 1), lambda i: (i, 0)),
            pl.BlockSpec((1, N), lambda i: (0, 0)),
            pl.BlockSpec((1, N), lambda i: (0, 0)),
            pl.BlockSpec((N, F_PRIME), lambda i: (0, 0)),
        ],
        out_specs=(
            pl.BlockSpec((ROW_BLOCK, F_PRIME), lambda i: (i, 0)),
            pl.BlockSpec((ROW_BLOCK, 1), lambda i: (i, 0)),
        ),
        out_shape=(
            jax.ShapeDtypeStruct((N, F_PRIME), jnp.float32),
            jax.ShapeDtypeStruct((N, 1), jnp.float32),
        ),
    )(A, P, p_sm, Q_row, q_row, z_src)

    den_sc = den_sc_flat.reshape(N, LAN)
    out = pl.pallas_call(
        _combine_kernel,
        out_shape=jax.ShapeDtypeStruct((N, F_PRIME), jnp.float32),
    )(num, den_tc, den_sc)
    return out


# speedup vs baseline: 1.4826x; 1.0810x over previous
import functools

import jax
import jax.numpy as jnp
from jax import lax
from jax.experimental import pallas as pl
from jax.experimental.pallas import tpu as pltpu
from jax.experimental.pallas import tpu_sc as plsc

N = 8192
F_IN = 128
F_PRIME = 64
NEG_SLOPE = 0.01

ROW_BLOCK = 512
SC_COLS = 1024
TC_COLS = N - SC_COLS
NW = 32
ROWS_PER_W = N // NW
RB = 16
LAN = 16


def _proj_kernel(h_ref, wst_ref, bs_ref, wtt_ref, bt_ref, a1_ref, a2_ref, ab_ref,
                 z_ref, P_ref, psm_ref, Q_ref, qsm_ref):
    h = h_ref[...]
    z_src = jnp.dot(h, wst_ref[...], preferred_element_type=jnp.float32) + bs_ref[...]
    z_tar = jnp.dot(h, wtt_ref[...], preferred_element_type=jnp.float32) + bt_ref[...]
    s = jnp.dot(z_src, a1_ref[...], preferred_element_type=jnp.float32) + ab_ref[...]
    t = jnp.dot(z_tar, a2_ref[...], preferred_element_type=jnp.float32)
    z_ref[...] = z_src
    P_ref[...] = jnp.exp(s)
    psm_ref[...] = jnp.exp(NEG_SLOPE * s)
    Q_ref[...] = jnp.exp(t)
    qsm_ref[...] = jnp.exp(NEG_SLOPE * t)


def _sc_den_kernel(a_hbm, P_hbm, p_hbm, Qt_hbm, qt_hbm, out_hbm,
                   qv, qqv, Pv, pv, abuf, dwide):
    c = lax.axis_index("c")
    s = lax.axis_index("s")
    wid = s * 2 + c
    row0 = wid * ROWS_PER_W
    pltpu.sync_copy(Qt_hbm, qv)
    pltpu.sync_copy(qt_hbm, qqv)
    pltpu.sync_copy(P_hbm.at[pl.ds(row0, ROWS_PER_W)], Pv)
    pltpu.sync_copy(p_hbm.at[pl.ds(row0, ROWS_PER_W)], pv)

    def blk_body(blk, carry):
        r0 = blk * RB
        pltpu.sync_copy(
            a_hbm.at[pl.ds(row0 + r0, RB), pl.ds(TC_COLS, SC_COLS)], abuf)
        Pgrp = Pv[pl.ds(r0, RB)]
        pgrp = pv[pl.ds(r0, RB)]
        Pbs = [jnp.full((LAN,), Pgrp[rr], jnp.float32) for rr in range(RB)]
        pbs = [jnp.full((LAN,), pgrp[rr], jnp.float32) for rr in range(RB)]
        zero = jnp.zeros((LAN,), jnp.float32)

        @plsc.parallel_loop(0, SC_COLS // LAN, carry=(zero,) * RB)
        def k_loop(k, accs):
            off = k * LAN
            qk = qv[pl.ds(off, LAN)]
            qqk = qqv[pl.ds(off, LAN)]
            new = []
            for rr in range(RB):
                a16 = abuf[rr, pl.ds(off, LAN)].astype(jnp.float32)
                e16 = jnp.maximum(Pbs[rr] * qk, pbs[rr] * qqk)
                new.append(accs[rr] + a16 * e16)
            return tuple(new)

        for rr in range(RB):
            dwide[pl.ds((r0 + rr) * LAN, LAN)] = k_loop[rr]
        return carry

    lax.fori_loop(0, ROWS_PER_W // RB, blk_body, jnp.int32(0))
    pltpu.sync_copy(dwide, out_hbm.at[pl.ds(row0 * LAN, ROWS_PER_W * LAN)])


def _agg_kernel(a_ref, P_ref, psm_ref, Q_ref, qsm_ref, z_ref, num_ref, den_ref):
    e = jnp.maximum(P_ref[...] * Q_ref[...], psm_ref[...] * qsm_ref[...])
    den_ref[...] = jnp.sum(e[:, :TC_COLS] * a_ref[...].astype(jnp.float32),
                           axis=1, keepdims=True)
    num_ref[...] = jnp.dot(e, z_ref[...], preferred_element_type=jnp.float32)


def _combine_kernel(num_ref, dtc_ref, dsc_ref, out_ref):
    den = dtc_ref[...] + jnp.sum(dsc_ref[...], axis=1, keepdims=True)
    out_ref[...] = jax.nn.sigmoid(num_ref[...] / den)


@jax.jit
def kernel(H, A, W_src_w, W_src_b, W_tar_w, W_tar_b, a_w, a_b):
    wst = W_src_w.T
    wtt = W_tar_w.T
    bs = W_src_b.reshape(1, F_PRIME)
    bt = W_tar_b.reshape(1, F_PRIME)
    a1 = a_w[:, :F_PRIME].T
    a2 = a_w[:, F_PRIME:].T
    ab = a_b.reshape(1, 1)

    z_src, P, p_sm, Q, q_sm = pl.pallas_call(
        _proj_kernel,
        out_shape=(
            jax.ShapeDtypeStruct((N, F_PRIME), jnp.float32),
            jax.ShapeDtypeStruct((N, 1), jnp.float32),
            jax.ShapeDtypeStruct((N, 1), jnp.float32),
            jax.ShapeDtypeStruct((N, 1), jnp.float32),
            jax.ShapeDtypeStruct((N, 1), jnp.float32),
        ),
    )(H, wst, bs, wtt, bt, a1, a2, ab)

    Q_row = Q.T
    q_row = q_sm.T
    P_1d = P.reshape(N)
    p_1d = p_sm.reshape(N)
    Qt = Q_row.reshape(N)[TC_COLS:]
    qt = q_row.reshape(N)[TC_COLS:]

    sc_mesh = plsc.VectorSubcoreMesh(core_axis_name="c", subcore_axis_name="s",
                                     num_cores=2, num_subcores=16)
    den_sc_flat = pl.kernel(
        _sc_den_kernel,
        out_type=jax.ShapeDtypeStruct((N * LAN,), jnp.float32),
        mesh=sc_mesh,
        scratch_types=[
            pltpu.VMEM((SC_COLS,), jnp.float32),
            pltpu.VMEM((SC_COLS,), jnp.float32),
            pltpu.VMEM((ROWS_PER_W,), jnp.float32),
            pltpu.VMEM((ROWS_PER_W,), jnp.float32),
            pltpu.VMEM((RB, SC_COLS), jnp.int32),
            pltpu.VMEM((ROWS_PER_W * LAN,), jnp.float32),
        ],
    )(A, P_1d, p_1d, Qt, qt)

    grid = (N // ROW_BLOCK,)
    num, den_tc = pl.pallas_call(
        _agg_kernel,
        grid=grid,
        in_specs=[
            pl.BlockSpec((ROW_BLOCK, TC_COLS), lambda i: (i, 0)),
            pl.BlockSpec((ROW_BLOCK, 1), lambda i: (i, 0)),
            pl.BlockSpec((ROW_BLOCK, 1), lambda i: (i, 0)),
            pl.BlockSpec((1, N), lambda i: (0, 0)),
            pl.BlockSpec((1, N), lambda i: (0, 0)),
            pl.BlockSpec((N, F_PRIME), lambda i: (0, 0)),
        ],
        out_specs=(
            pl.BlockSpec((ROW_BLOCK, F_PRIME), lambda i: (i, 0)),
            pl.BlockSpec((ROW_BLOCK, 1), lambda i: (i, 0)),
        ),
        out_shape=(
            jax.ShapeDtypeStruct((N, F_PRIME), jnp.float32),
            jax.ShapeDtypeStruct((N, 1), jnp.float32),
        ),
    )(A, P, p_sm, Q_row, q_row, z_src)

    den_sc = den_sc_flat.reshape(N, LAN)
    out = pl.pallas_call(
        _combine_kernel,
        out_shape=jax.ShapeDtypeStruct((N, F_PRIME), jnp.float32),
    )(num, den_tc, den_sc)
    return out


# speedup vs baseline: 1.8867x; 1.2725x over previous
import functools

import jax
import jax.numpy as jnp
from jax.experimental import pallas as pl

N = 8192
F_IN = 128
F_PRIME = 64
NEG_SLOPE = 0.01

ROW_BLOCK = 512
N_CHUNKS = 2


def _proj_kernel(h_ref, wst_ref, bs_ref, wtt_ref, bt_ref, a1_ref, a2_ref, ab_ref,
                 z_ref, P_ref, psm_ref, Q_ref, qsm_ref):
    h = h_ref[...]
    z_src = jnp.dot(h, wst_ref[...], preferred_element_type=jnp.float32) + bs_ref[...]
    z_tar = jnp.dot(h, wtt_ref[...], preferred_element_type=jnp.float32) + bt_ref[...]
    s = jnp.dot(z_src, a1_ref[...], preferred_element_type=jnp.float32) + ab_ref[...]
    t = jnp.dot(z_tar, a2_ref[...], preferred_element_type=jnp.float32)
    z_ref[...] = z_src
    P_ref[...] = jnp.exp(s)
    psm_ref[...] = jnp.exp(NEG_SLOPE * s)
    Q_ref[...] = jnp.exp(t)
    qsm_ref[...] = jnp.exp(NEG_SLOPE * t)


def _agg_kernel(*refs):
    a_refs = refs[:N_CHUNKS]
    P_ref, psm_ref, Q_ref, qsm_ref, z_ref, out_ref = refs[N_CHUNKS:]
    C = N // N_CHUNKS
    P = P_ref[...]
    psm = psm_ref[...]
    den = None
    num = None
    for c in range(N_CHUNKS):
        e = jnp.maximum(P * Q_ref[:, c * C:(c + 1) * C],
                        psm * qsm_ref[:, c * C:(c + 1) * C])
        d = jnp.sum(e * a_refs[c][...].astype(jnp.float32), axis=1, keepdims=True)
        n = jnp.dot(e, z_ref[c * C:(c + 1) * C, :],
                    preferred_element_type=jnp.float32)
        den = d if den is None else den + d
        num = n if num is None else num + n
    out_ref[...] = jax.nn.sigmoid(num / den)


@jax.jit
def kernel(H, A, W_src_w, W_src_b, W_tar_w, W_tar_b, a_w, a_b):
    wst = W_src_w.T
    wtt = W_tar_w.T
    bs = W_src_b.reshape(1, F_PRIME)
    bt = W_tar_b.reshape(1, F_PRIME)
    a1 = a_w[:, :F_PRIME].T
    a2 = a_w[:, F_PRIME:].T
    ab = a_b.reshape(1, 1)

    z_src, P, p_sm, Q, q_sm = pl.pallas_call(
        _proj_kernel,
        out_shape=(
            jax.ShapeDtypeStruct((N, F_PRIME), jnp.float32),
            jax.ShapeDtypeStruct((N, 1), jnp.float32),
            jax.ShapeDtypeStruct((N, 1), jnp.float32),
            jax.ShapeDtypeStruct((N, 1), jnp.float32),
            jax.ShapeDtypeStruct((N, 1), jnp.float32),
        ),
    )(H, wst, bs, wtt, bt, a1, a2, ab)

    Q_row = Q.T
    q_row = q_sm.T

    grid = (N // ROW_BLOCK,)
    out = pl.pallas_call(
        _agg_kernel,
        grid=grid,
        in_specs=[
            pl.BlockSpec((ROW_BLOCK, N // N_CHUNKS),
                         functools.partial(lambda c, i: (i, c), c))
            for c in range(N_CHUNKS)
        ] + [
            pl.BlockSpec((ROW_BLOCK, 1), lambda i: (i, 0)),
            pl.BlockSpec((ROW_BLOCK, 1), lambda i: (i, 0)),
            pl.BlockSpec((1, N), lambda i: (0, 0)),
            pl.BlockSpec((1, N), lambda i: (0, 0)),
            pl.BlockSpec((N, F_PRIME), lambda i: (0, 0)),
        ],
        out_specs=pl.BlockSpec((ROW_BLOCK, F_PRIME), lambda i: (i, 0)),
        out_shape=jax.ShapeDtypeStruct((N, F_PRIME), jnp.float32),
    )(*([A] * N_CHUNKS), P, p_sm, Q_row, q_row, z_src)
    return out


# speedup vs baseline: 1.8905x; 1.0020x over previous
import functools

import jax
import jax.numpy as jnp
from jax.experimental import pallas as pl

N = 8192
F_IN = 128
F_PRIME = 64
NEG_SLOPE = 0.01

ROW_BLOCK = 512
N_CHUNKS = 4


def _proj_kernel(h_ref, wst_ref, bs_ref, wtt_ref, bt_ref, a1_ref, a2_ref, ab_ref,
                 z_ref, P_ref, psm_ref, Q_ref, qsm_ref):
    h = h_ref[...]
    z_src = jnp.dot(h, wst_ref[...], preferred_element_type=jnp.float32) + bs_ref[...]
    z_tar = jnp.dot(h, wtt_ref[...], preferred_element_type=jnp.float32) + bt_ref[...]
    s = jnp.dot(z_src, a1_ref[...], preferred_element_type=jnp.float32) + ab_ref[...]
    t = jnp.dot(z_tar, a2_ref[...], preferred_element_type=jnp.float32)
    z_ref[...] = z_src
    P_ref[...] = jnp.exp(s)
    psm_ref[...] = jnp.exp(NEG_SLOPE * s)
    Q_ref[...] = jnp.exp(t)
    qsm_ref[...] = jnp.exp(NEG_SLOPE * t)


def _agg_kernel(*refs):
    a_refs = refs[:N_CHUNKS]
    P_ref, psm_ref, Q_ref, qsm_ref, z_ref, out_ref = refs[N_CHUNKS:]
    C = N // N_CHUNKS
    P = P_ref[...]
    psm = psm_ref[...]
    den = None
    num = None
    for c in range(N_CHUNKS):
        e = jnp.maximum(P * Q_ref[:, c * C:(c + 1) * C],
                        psm * qsm_ref[:, c * C:(c + 1) * C])
        d = jnp.sum(e * a_refs[c][...].astype(jnp.float32), axis=1, keepdims=True)
        n = jnp.dot(e, z_ref[c * C:(c + 1) * C, :],
                    preferred_element_type=jnp.float32)
        den = d if den is None else den + d
        num = n if num is None else num + n
    out_ref[...] = jax.nn.sigmoid(num / den)


@jax.jit
def kernel(H, A, W_src_w, W_src_b, W_tar_w, W_tar_b, a_w, a_b):
    wst = W_src_w.T
    wtt = W_tar_w.T
    bs = W_src_b.reshape(1, F_PRIME)
    bt = W_tar_b.reshape(1, F_PRIME)
    a1 = a_w[:, :F_PRIME].T
    a2 = a_w[:, F_PRIME:].T
    ab = a_b.reshape(1, 1)

    z_src, P, p_sm, Q, q_sm = pl.pallas_call(
        _proj_kernel,
        out_shape=(
            jax.ShapeDtypeStruct((N, F_PRIME), jnp.float32),
            jax.ShapeDtypeStruct((N, 1), jnp.float32),
            jax.ShapeDtypeStruct((N, 1), jnp.float32),
            jax.ShapeDtypeStruct((N, 1), jnp.float32),
            jax.ShapeDtypeStruct((N, 1), jnp.float32),
        ),
    )(H, wst, bs, wtt, bt, a1, a2, ab)

    Q_row = Q.T
    q_row = q_sm.T

    grid = (N // ROW_BLOCK,)
    out = pl.pallas_call(
        _agg_kernel,
        grid=grid,
        in_specs=[
            pl.BlockSpec((ROW_BLOCK, N // N_CHUNKS),
                         functools.partial(lambda c, i: (i, c), c))
            for c in range(N_CHUNKS)
        ] + [
            pl.BlockSpec((ROW_BLOCK, 1), lambda i: (i, 0)),
            pl.BlockSpec((ROW_BLOCK, 1), lambda i: (i, 0)),
            pl.BlockSpec((1, N), lambda i: (0, 0)),
            pl.BlockSpec((1, N), lambda i: (0, 0)),
            pl.BlockSpec((N, F_PRIME), lambda i: (0, 0)),
        ],
        out_specs=pl.BlockSpec((ROW_BLOCK, F_PRIME), lambda i: (i, 0)),
        out_shape=jax.ShapeDtypeStruct((N, F_PRIME), jnp.float32),
    )(*([A] * N_CHUNKS), P, p_sm, Q_row, q_row, z_src)
    return out


# speedup vs baseline: 1.9280x; 1.0199x over previous
import functools

import jax
import jax.numpy as jnp
from jax.experimental import pallas as pl

N = 8192
F_IN = 128
F_PRIME = 64
NEG_SLOPE = 0.01

ROW_BLOCK = 512
N_CHUNKS = 8


def _proj_kernel(h_ref, wst_ref, bs_ref, wtt_ref, bt_ref, a1_ref, a2_ref, ab_ref,
                 z_ref, P_ref, psm_ref, Q_ref, qsm_ref):
    h = h_ref[...]
    z_src = jnp.dot(h, wst_ref[...], preferred_element_type=jnp.float32) + bs_ref[...]
    z_tar = jnp.dot(h, wtt_ref[...], preferred_element_type=jnp.float32) + bt_ref[...]
    s = jnp.dot(z_src, a1_ref[...], preferred_element_type=jnp.float32) + ab_ref[...]
    t = jnp.dot(z_tar, a2_ref[...], preferred_element_type=jnp.float32)
    z_ref[...] = z_src
    P_ref[...] = jnp.exp(s)
    psm_ref[...] = jnp.exp(NEG_SLOPE * s)
    Q_ref[...] = jnp.exp(t)
    qsm_ref[...] = jnp.exp(NEG_SLOPE * t)


def _agg_kernel(*refs):
    a_refs = refs[:N_CHUNKS]
    P_ref, psm_ref, Q_ref, qsm_ref, z_ref, out_ref = refs[N_CHUNKS:]
    C = N // N_CHUNKS
    P = P_ref[...]
    psm = psm_ref[...]
    den = None
    num = None
    for c in range(N_CHUNKS):
        e = jnp.maximum(P * Q_ref[:, c * C:(c + 1) * C],
                        psm * qsm_ref[:, c * C:(c + 1) * C])
        d = jnp.sum(e * a_refs[c][...].astype(jnp.float32), axis=1, keepdims=True)
        n = jnp.dot(e, z_ref[c * C:(c + 1) * C, :],
                    preferred_element_type=jnp.float32)
        den = d if den is None else den + d
        num = n if num is None else num + n
    out_ref[...] = jax.nn.sigmoid(num / den)


@jax.jit
def kernel(H, A, W_src_w, W_src_b, W_tar_w, W_tar_b, a_w, a_b):
    wst = W_src_w.T
    wtt = W_tar_w.T
    bs = W_src_b.reshape(1, F_PRIME)
    bt = W_tar_b.reshape(1, F_PRIME)
    a1 = a_w[:, :F_PRIME].T
    a2 = a_w[:, F_PRIME:].T
    ab = a_b.reshape(1, 1)

    z_src, P, p_sm, Q, q_sm = pl.pallas_call(
        _proj_kernel,
        out_shape=(
            jax.ShapeDtypeStruct((N, F_PRIME), jnp.float32),
            jax.ShapeDtypeStruct((N, 1), jnp.float32),
            jax.ShapeDtypeStruct((N, 1), jnp.float32),
            jax.ShapeDtypeStruct((N, 1), jnp.float32),
            jax.ShapeDtypeStruct((N, 1), jnp.float32),
        ),
    )(H, wst, bs, wtt, bt, a1, a2, ab)

    Q_row = Q.T
    q_row = q_sm.T

    grid = (N // ROW_BLOCK,)
    out = pl.pallas_call(
        _agg_kernel,
        grid=grid,
        in_specs=[
            pl.BlockSpec((ROW_BLOCK, N // N_CHUNKS),
                         functools.partial(lambda c, i: (i, c), c))
            for c in range(N_CHUNKS)
        ] + [
            pl.BlockSpec((ROW_BLOCK, 1), lambda i: (i, 0)),
            pl.BlockSpec((ROW_BLOCK, 1), lambda i: (i, 0)),
            pl.BlockSpec((1, N), lambda i: (0, 0)),
            pl.BlockSpec((1, N), lambda i: (0, 0)),
            pl.BlockSpec((N, F_PRIME), lambda i: (0, 0)),
        ],
        out_specs=pl.BlockSpec((ROW_BLOCK, F_PRIME), lambda i: (i, 0)),
        out_shape=jax.ShapeDtypeStruct((N, F_PRIME), jnp.float32),
    )(*([A] * N_CHUNKS), P, p_sm, Q_row, q_row, z_src)
    return out


# speedup vs baseline: 1.9909x; 1.0326x over previous
import functools

import jax
import jax.numpy as jnp
from jax.experimental import pallas as pl

N = 8192
F_IN = 128
F_PRIME = 64
NEG_SLOPE = 0.01

ROW_BLOCK = 512
N_CHUNKS = 16


def _proj_kernel(h_ref, wst_ref, bs_ref, wtt_ref, bt_ref, a1_ref, a2_ref, ab_ref,
                 z_ref, P_ref, psm_ref, Q_ref, qsm_ref):
    h = h_ref[...]
    z_src = jnp.dot(h, wst_ref[...], preferred_element_type=jnp.float32) + bs_ref[...]
    z_tar = jnp.dot(h, wtt_ref[...], preferred_element_type=jnp.float32) + bt_ref[...]
    s = jnp.dot(z_src, a1_ref[...], preferred_element_type=jnp.float32) + ab_ref[...]
    t = jnp.dot(z_tar, a2_ref[...], preferred_element_type=jnp.float32)
    z_ref[...] = z_src
    P_ref[...] = jnp.exp(s)
    psm_ref[...] = jnp.exp(NEG_SLOPE * s)
    Q_ref[...] = jnp.exp(t)
    qsm_ref[...] = jnp.exp(NEG_SLOPE * t)


def _agg_kernel(*refs):
    a_refs = refs[:N_CHUNKS]
    P_ref, psm_ref, Q_ref, qsm_ref, z_ref, out_ref = refs[N_CHUNKS:]
    C = N // N_CHUNKS
    P = P_ref[...]
    psm = psm_ref[...]
    den = None
    num = None
    for c in range(N_CHUNKS):
        e = jnp.maximum(P * Q_ref[:, c * C:(c + 1) * C],
                        psm * qsm_ref[:, c * C:(c + 1) * C])
        d = jnp.sum(e * a_refs[c][...].astype(jnp.float32), axis=1, keepdims=True)
        n = jnp.dot(e, z_ref[c * C:(c + 1) * C, :],
                    preferred_element_type=jnp.float32)
        den = d if den is None else den + d
        num = n if num is None else num + n
    out_ref[...] = jax.nn.sigmoid(num / den)


@jax.jit
def kernel(H, A, W_src_w, W_src_b, W_tar_w, W_tar_b, a_w, a_b):
    wst = W_src_w.T
    wtt = W_tar_w.T
    bs = W_src_b.reshape(1, F_PRIME)
    bt = W_tar_b.reshape(1, F_PRIME)
    a1 = a_w[:, :F_PRIME].T
    a2 = a_w[:, F_PRIME:].T
    ab = a_b.reshape(1, 1)

    z_src, P, p_sm, Q, q_sm = pl.pallas_call(
        _proj_kernel,
        out_shape=(
            jax.ShapeDtypeStruct((N, F_PRIME), jnp.float32),
            jax.ShapeDtypeStruct((N, 1), jnp.float32),
            jax.ShapeDtypeStruct((N, 1), jnp.float32),
            jax.ShapeDtypeStruct((N, 1), jnp.float32),
            jax.ShapeDtypeStruct((N, 1), jnp.float32),
        ),
    )(H, wst, bs, wtt, bt, a1, a2, ab)

    Q_row = Q.T
    q_row = q_sm.T

    grid = (N // ROW_BLOCK,)
    out = pl.pallas_call(
        _agg_kernel,
        grid=grid,
        in_specs=[
            pl.BlockSpec((ROW_BLOCK, N // N_CHUNKS),
                         functools.partial(lambda c, i: (i, c), c))
            for c in range(N_CHUNKS)
        ] + [
            pl.BlockSpec((ROW_BLOCK, 1), lambda i: (i, 0)),
            pl.BlockSpec((ROW_BLOCK, 1), lambda i: (i, 0)),
            pl.BlockSpec((1, N), lambda i: (0, 0)),
            pl.BlockSpec((1, N), lambda i: (0, 0)),
            pl.BlockSpec((N, F_PRIME), lambda i: (0, 0)),
        ],
        out_specs=pl.BlockSpec((ROW_BLOCK, F_PRIME), lambda i: (i, 0)),
        out_shape=jax.ShapeDtypeStruct((N, F_PRIME), jnp.float32),
    )(*([A] * N_CHUNKS), P, p_sm, Q_row, q_row, z_src)
    return out


# speedup vs baseline: 2.0617x; 1.0355x over previous
import functools

import jax
import jax.numpy as jnp
from jax.experimental import pallas as pl

N = 8192
F_IN = 128
F_PRIME = 64
NEG_SLOPE = 0.01

ROW_BLOCK = 512
N_CHUNKS = 32


def _proj_kernel(h_ref, wst_ref, bs_ref, wtt_ref, bt_ref, a1_ref, a2_ref, ab_ref,
                 z_ref, P_ref, psm_ref, Q_ref, qsm_ref):
    h = h_ref[...]
    z_src = jnp.dot(h, wst_ref[...], preferred_element_type=jnp.float32) + bs_ref[...]
    z_tar = jnp.dot(h, wtt_ref[...], preferred_element_type=jnp.float32) + bt_ref[...]
    s = jnp.dot(z_src, a1_ref[...], preferred_element_type=jnp.float32) + ab_ref[...]
    t = jnp.dot(z_tar, a2_ref[...], preferred_element_type=jnp.float32)
    z_ref[...] = z_src
    P_ref[...] = jnp.exp(s)
    psm_ref[...] = jnp.exp(NEG_SLOPE * s)
    Q_ref[...] = jnp.exp(t)
    qsm_ref[...] = jnp.exp(NEG_SLOPE * t)


def _agg_kernel(*refs):
    a_refs = refs[:N_CHUNKS]
    P_ref, psm_ref, Q_ref, qsm_ref, z_ref, out_ref = refs[N_CHUNKS:]
    C = N // N_CHUNKS
    P = P_ref[...]
    psm = psm_ref[...]
    den = None
    num = None
    for c in range(N_CHUNKS):
        e = jnp.maximum(P * Q_ref[:, c * C:(c + 1) * C],
                        psm * qsm_ref[:, c * C:(c + 1) * C])
        d = jnp.sum(e * a_refs[c][...].astype(jnp.float32), axis=1, keepdims=True)
        n = jnp.dot(e, z_ref[c * C:(c + 1) * C, :],
                    preferred_element_type=jnp.float32)
        den = d if den is None else den + d
        num = n if num is None else num + n
    out_ref[...] = jax.nn.sigmoid(num / den)


@jax.jit
def kernel(H, A, W_src_w, W_src_b, W_tar_w, W_tar_b, a_w, a_b):
    wst = W_src_w.T
    wtt = W_tar_w.T
    bs = W_src_b.reshape(1, F_PRIME)
    bt = W_tar_b.reshape(1, F_PRIME)
    a1 = a_w[:, :F_PRIME].T
    a2 = a_w[:, F_PRIME:].T
    ab = a_b.reshape(1, 1)

    z_src, P, p_sm, Q, q_sm = pl.pallas_call(
        _proj_kernel,
        out_shape=(
            jax.ShapeDtypeStruct((N, F_PRIME), jnp.float32),
            jax.ShapeDtypeStruct((N, 1), jnp.float32),
            jax.ShapeDtypeStruct((N, 1), jnp.float32),
            jax.ShapeDtypeStruct((N, 1), jnp.float32),
            jax.ShapeDtypeStruct((N, 1), jnp.float32),
        ),
    )(H, wst, bs, wtt, bt, a1, a2, ab)

    Q_row = Q.T
    q_row = q_sm.T

    grid = (N // ROW_BLOCK,)
    out = pl.pallas_call(
        _agg_kernel,
        grid=grid,
        in_specs=[
            pl.BlockSpec((ROW_BLOCK, N // N_CHUNKS),
                         functools.partial(lambda c, i: (i, c), c))
            for c in range(N_CHUNKS)
        ] + [
            pl.BlockSpec((ROW_BLOCK, 1), lambda i: (i, 0)),
            pl.BlockSpec((ROW_BLOCK, 1), lambda i: (i, 0)),
            pl.BlockSpec((1, N), lambda i: (0, 0)),
            pl.BlockSpec((1, N), lambda i: (0, 0)),
            pl.BlockSpec((N, F_PRIME), lambda i: (0, 0)),
        ],
        out_specs=pl.BlockSpec((ROW_BLOCK, F_PRIME), lambda i: (i, 0)),
        out_shape=jax.ShapeDtypeStruct((N, F_PRIME), jnp.float32),
    )(*([A] * N_CHUNKS), P, p_sm, Q_row, q_row, z_src)
    return out
